# baseline (speedup 1.0000x reference)
import functools

import jax
import jax.numpy as jnp
from jax import lax
from jax.experimental import pallas as pl
from jax.experimental.pallas import tpu as pltpu
from jax.experimental.pallas import tpu_sc as plsc

B = 16384
F = 26
V = 100000
VP = 100096
NC = 2
NS = 16
FH = F // 2
SPT = B // NS
W = 768
TAIL = 99968


IMG = FH * VP


def _sc_body(table_hbm, x_hbm, tail_hbm, out_hbm, img_hbm,
             tblk_v, tblk2_v, tailb_v, row_v, xblk_v, idx_v, rows_v, out_v,
             sem, gsem, rsem, bsem):
    sc = lax.axis_index("c")
    s = lax.axis_index("s")
    ibase = sc * IMG

    pltpu.sync_copy(x_hbm.at[:, pl.ds(s * SPT, SPT)], xblk_v)

    def extract_rows(buf, f0, width, dst_col):
        for fl in range(FH):
            def chunk(c, _, fl=fl):
                for k in range(4):
                    cc = c * 64 + k * 16
                    row_v[pl.ds(fl * W + cc, 16)] = buf[f0 + fl,
                                                        pl.ds(cc, 16)]
                return 0
            lax.fori_loop(0, width // 64, chunk, 0)
        for fl in range(FH):
            pltpu.make_async_copy(
                row_v.at[pl.ds(fl * W, width)],
                img_hbm.at[pl.ds(ibase + fl * VP + dst_col, width)],
                rsem).start()

    def drain_rows(width):
        pltpu.make_async_copy(
            img_hbm.at[pl.ds(0, FH * width)],
            row_v.at[pl.ds(0, FH * width)],
            rsem).wait()

    def stage_all(f0):
        base_c = s * (8 * W)

        def bdma(buf, c0):
            return pltpu.make_async_copy(
                table_hbm.at[:, pl.ds(c0, W)], buf, bsem)

        def blk(g, _):
            c_a = base_c + (2 * g) * W
            c_b = c_a + W
            bdma(tblk2_v, c_b).start()
            bdma(tblk_v, c_a).wait()
            drain_rows(W)
            extract_rows(tblk_v, f0, W, c_a)

            @pl.when(g < 3)
            def _():
                bdma(tblk_v, c_b + W).start()

            bdma(tblk2_v, c_b).wait()
            drain_rows(W)
            extract_rows(tblk2_v, f0, W, c_b)
            return 0

        bdma(tblk_v, base_c).start()
        for fl in range(FH):
            pltpu.make_async_copy(
                row_v.at[pl.ds(fl * W, W)],
                img_hbm.at[pl.ds(ibase + fl * VP + base_c, W)],
                rsem).start()
        lax.fori_loop(0, 4, blk, 0)

        @pl.when(s == NS - 1)
        def _():
            def blk2(j, _):
                c0 = 98304 + j * W
                pltpu.sync_copy(table_hbm.at[:, pl.ds(c0, W)],
                                tblk_v.at[:, pl.ds(0, W)])
                drain_rows(W)
                extract_rows(tblk_v, f0, W, c0)
                return 0

            lax.fori_loop(0, 2, blk2, 0)
            c0 = 99840
            pltpu.sync_copy(table_hbm.at[:, pl.ds(c0, 128)],
                            tblk_v.at[:, pl.ds(0, 128)])
            drain_rows(W)
            extract_rows(tblk_v, f0, 128, c0)
            drain_rows(128)
            pltpu.sync_copy(tail_hbm, tailb_v)
            for fl in range(FH):
                def tchunk(c, _, fl=fl):
                    row_v[pl.ds(c * 16, 16)] = tailb_v[f0 + fl,
                                                       pl.ds(c * 16, 16)]
                    return 0
                lax.fori_loop(0, 2, tchunk, 0)
                pltpu.sync_copy(row_v.at[pl.ds(0, 32)],
                                img_hbm.at[pl.ds(ibase + fl * VP + TAIL,
                                                 32)])

        @pl.when(s < NS - 1)
        def _():
            drain_rows(W)

    def build_idx(f0):
        def chunk(c, _):
            for fl in range(FH):
                x16 = xblk_v[f0 + fl, pl.ds(c * 16, 16)]
                idx_v[pl.ds(fl * SPT + c * 16, 16)] = (
                    x16 + jnp.int32(fl * VP)) + ibase
            return 0
        lax.fori_loop(0, SPT // 16, chunk, 0)

    @pl.when(sc == 0)
    def _():
        stage_all(0)
        build_idx(0)

    @pl.when(sc == 1)
    def _():
        stage_all(FH)
        build_idx(FH)

    plsc.subcore_barrier()

    pltpu.async_copy(img_hbm.at[idx_v], rows_v, gsem).wait()

    def reduce_chunk(c, _):
        acc = rows_v[pl.ds(c * 16, 16)]
        for fl in range(1, FH):
            acc = acc + rows_v[pl.ds(fl * SPT + c * 16, 16)]
        out_v[pl.ds(c * 16, 16)] = acc
        return 0

    lax.fori_loop(0, SPT // 16, reduce_chunk, 0)

    pltpu.sync_copy(out_v, out_hbm.at[pl.ds(sc * B + s * SPT, SPT)])


_sc_kernel = functools.partial(
    pl.kernel,
    out_type=(jax.ShapeDtypeStruct((NC * B,), jnp.float32),
              jax.ShapeDtypeStruct((NC * FH * VP,), jnp.float32)),
    mesh=plsc.VectorSubcoreMesh(
        core_axis_name="c", subcore_axis_name="s",
        num_cores=NC, num_subcores=NS),
    compiler_params=pltpu.CompilerParams(
        needs_layout_passes=False, use_tc_tiling_on_sc=True),
    scratch_types=[
        pltpu.VMEM((F, W), jnp.float32),
        pltpu.VMEM((F, W), jnp.float32),
        pltpu.VMEM((F, 32), jnp.float32),
        pltpu.VMEM((FH * W,), jnp.float32),
        pltpu.VMEM((F, SPT), jnp.int32),
        pltpu.VMEM((FH * SPT,), jnp.int32),
        pltpu.VMEM((FH * SPT,), jnp.float32),
        pltpu.VMEM((SPT,), jnp.float32),
        pltpu.SemaphoreType.DMA,
        pltpu.SemaphoreType.DMA,
        pltpu.SemaphoreType.DMA,
        pltpu.SemaphoreType.DMA,
    ],
)(_sc_body)


def kernel(X, emb_tables):
    table_2d = emb_tables.reshape(F, V)
    tail = emb_tables[:, TAIL:, 0]
    partials, _ = _sc_kernel(table_2d, X.T, tail)
    p = partials.reshape(NC, B)
    return jax.nn.sigmoid(p[0] + p[1]).reshape(B, 1)


# speedup vs baseline: 1.4359x; 1.4359x over previous
import functools

import jax
import jax.numpy as jnp
from jax import lax
from jax.experimental import pallas as pl
from jax.experimental.pallas import tpu as pltpu
from jax.experimental.pallas import tpu_sc as plsc

B = 16384
F = 26
V = 100000
NC = 2
NS = 16
NW = NC * NS
BPW = B // NW
EPW = BPW * F


def _sc_body(table_hbm, x_hbm, out_hbm, xblk_v, idx_v, rows_v, out_v,
             sem, gsem):
    wid = lax.axis_index("s") * NC + lax.axis_index("c")
    base = wid * BPW

    pltpu.sync_copy(x_hbm.at[:, pl.ds(base, BPW)], xblk_v)

    def build(c, _):
        for f in range(F):
            x16 = xblk_v[f, pl.ds(c * 16, 16)]
            idx_v[pl.ds(f * BPW + c * 16, 16)] = x16 + jnp.int32(f * V)
        return 0

    lax.fori_loop(0, BPW // 16, build, 0)

    pltpu.async_copy(table_hbm.at[idx_v], rows_v, gsem).wait()

    def reduce_chunk(c, _):
        acc = rows_v[pl.ds(c * 16, 16)]
        for f in range(1, F):
            acc = acc + rows_v[pl.ds(f * BPW + c * 16, 16)]
        out_v[pl.ds(c * 16, 16)] = 1.0 / (1.0 + jnp.exp(-acc))
        return 0

    lax.fori_loop(0, BPW // 16, reduce_chunk, 0)

    pltpu.sync_copy(out_v, out_hbm.at[pl.ds(base, BPW)])


_sc_kernel = functools.partial(
    pl.kernel,
    out_type=jax.ShapeDtypeStruct((B,), jnp.float32),
    mesh=plsc.VectorSubcoreMesh(
        core_axis_name="c", subcore_axis_name="s",
        num_cores=NC, num_subcores=NS),
    compiler_params=pltpu.CompilerParams(
        needs_layout_passes=False, use_tc_tiling_on_sc=True),
    scratch_types=[
        pltpu.VMEM((F, BPW), jnp.int32),
        pltpu.VMEM((EPW,), jnp.int32),
        pltpu.VMEM((EPW,), jnp.float32),
        pltpu.VMEM((BPW,), jnp.float32),
        pltpu.SemaphoreType.DMA,
        pltpu.SemaphoreType.DMA,
    ],
)(_sc_body)


def kernel(X, emb_tables):
    t2 = jnp.squeeze(emb_tables, 2)
    t2 = lax.optimization_barrier(t2)
    table_flat = t2.reshape(F * V)
    out = _sc_kernel(table_flat, X.T)
    return out.reshape(B, 1)
